# Initial kernel scaffold; baseline (speedup 1.0000x reference)
import jax
import jax.numpy as jnp
from jax.experimental import pallas as pl


def kernel(node, edge, edge_index, node_index, coupling_index, W_nemb, b_nemb, W_eemb, b_eemb, enc1_W0, enc1_W1, enc1_W2, enc2_W0, enc2_W1, enc3_W0, enc3_W1, p1_W, p1_b, p1_g, p1_beta, p2_W, p2_b, p2_g, p2_beta, p3_W, p3_b):
    raise NotImplementedError("write your pallas kernel here")


# speedup vs baseline: 1.6710x; 1.6710x over previous
import functools

import jax
import jax.numpy as jnp
from jax import lax
from jax.experimental import pallas as pl
from jax.experimental.pallas import tpu as pltpu
from jax.experimental.pallas import tpu_sc as plsc

_NC, _NS = 2, 16
_NW = _NC * _NS
_CHUNK = 128
_G = 512
_RS = float((1.0 + 1e-5) ** -0.5)
_f32 = jnp.float32


def _whole(shape):
  return pl.BlockSpec(shape, lambda i: (0,) * len(shape))


def _prep(node, W_nemb, b_nemb, W_eemb, b_eemb, e1W2, p1We, p1b):
  n, d = node.shape
  de = W_eemb.shape[0]
  h1 = p1We.shape[1]
  bn = 1000
  grid = n // bn

  def body(node_ref, wn_ref, bnemb_ref, we_ref, beemb_ref, e1w2_ref, p1we_ref,
           p1b_ref, nh0_ref, wfaug_ref, weh_ref, b1eff_ref):
    x = node_ref[...]
    nh0_ref[...] = (jnp.dot(x, wn_ref[...], preferred_element_type=_f32, precision=lax.Precision.HIGHEST)
                    + bnemb_ref[...][None, :])

    @pl.when(pl.program_id(0) == 0)
    def _():
      we = we_ref[...]
      e1w2 = e1w2_ref[...]
      p1we = p1we_ref[...]
      be = beemb_ref[...]
      wfaug_ref[...] = jnp.zeros((d, d), _f32)
      wfaug_ref[0:de, :] = jnp.dot(we, e1w2, preferred_element_type=_f32, precision=lax.Precision.HIGHEST)
      bf = jnp.sum(be[:, None] * e1w2, axis=0)
      wfaug_ref[de:de + 1, :] = bf[None, :]
      weh_ref[...] = jnp.dot(we, p1we, preferred_element_type=_f32, precision=lax.Precision.HIGHEST)
      b1 = p1b_ref[...] + jnp.sum(be[:, None] * p1we, axis=0)
      b1eff_ref[...] = jnp.broadcast_to(b1[None, :], (8, h1))

  return pl.pallas_call(
      body,
      grid=(grid,),
      in_specs=[
          pl.BlockSpec((bn, d), lambda i: (i, 0)),
          _whole((d, d)), _whole((d,)), _whole((de, d)), _whole((d,)),
          _whole((d, d)), _whole((d, h1)), _whole((h1,)),
      ],
      out_specs=(
          pl.BlockSpec((bn, d), lambda i: (i, 0)),
          _whole((d, d)), _whole((de, h1)), _whole((8, h1)),
      ),
      out_shape=(
          jax.ShapeDtypeStruct((n, d), _f32),
          jax.ShapeDtypeStruct((d, d), _f32),
          jax.ShapeDtypeStruct((de, h1), _f32),
          jax.ShapeDtypeStruct((8, h1), _f32),
      ),
  )(node, W_nemb, b_nemb, W_eemb, b_eemb, e1W2, p1We, p1b)


def _sc_mesh():
  return plsc.VectorSubcoreMesh(core_axis_name="c", subcore_axis_name="s",
                                num_cores=_NC, num_subcores=_NS)


def _rows_split(n):
  rps = (-(-n // _NS) + 7) // 8 * 8
  last = n - (_NS - 1) * rps
  return rps, last


def _sc_edge_prep(sd, edge_aug, z128, n):
  d = edge_aug.shape[1]
  nchunks = sd.shape[0]
  rps, rlast = _rows_split(n)

  @functools.partial(
      pl.kernel, mesh=_sc_mesh(),
      out_type=jax.ShapeDtypeStruct((_NC, n, d), _f32),
      scratch_types=[
          pltpu.VMEM((2, _CHUNK), jnp.int32),
          pltpu.VMEM((_CHUNK, d), _f32),
          pltpu.VMEM_SHARED((n, d), _f32),
      ],
  )
  def k(sd_hbm, ea_hbm, z128_hbm, out_hbm, idxb, erows, acc):
    c = lax.axis_index("c")
    s = lax.axis_index("s")
    w = s * _NC + c
    r0 = pl.multiple_of(s * rps, 8)
    pl.when(s < _NS - 1)(lambda: pltpu.sync_copy(
        z128_hbm.at[pl.ds(r0, rps)], acc.at[pl.ds(r0, rps)]))
    pl.when(s == _NS - 1)(lambda: pltpu.sync_copy(
        z128_hbm.at[pl.ds(r0, rlast)], acc.at[pl.ds(r0, rlast)]))
    plsc.subcore_barrier()
    lo = nchunks * w // _NW
    hi = nchunks * (w + 1) // _NW

    def step(i, carry):
      pltpu.sync_copy(sd_hbm.at[i], idxb)
      pltpu.sync_copy(ea_hbm.at[pl.ds(i * _CHUNK, _CHUNK)], erows)
      pltpu.sync_copy(erows, acc.at[idxb.at[1]], add=True)
      return carry

    lax.fori_loop(lo, hi, step, 0)
    plsc.subcore_barrier()
    pl.when(s < _NS - 1)(lambda: pltpu.sync_copy(
        acc.at[pl.ds(r0, rps)], out_hbm.at[c, pl.ds(r0, rps)]))
    pl.when(s == _NS - 1)(lambda: pltpu.sync_copy(
        acc.at[pl.ds(r0, rlast)], out_hbm.at[c, pl.ds(r0, rlast)]))

  return k(sd, edge_aug, z128)


def _sc_scatter(nh, sd, z128):
  n, d = nh.shape
  nchunks = sd.shape[0]
  rps, rlast = _rows_split(n)

  @functools.partial(
      pl.kernel, mesh=_sc_mesh(),
      out_type=jax.ShapeDtypeStruct((_NC, n, d), _f32),
      scratch_types=[
          pltpu.VMEM((2, _CHUNK), jnp.int32),
          pltpu.VMEM((_CHUNK, d), _f32),
          pltpu.VMEM_SHARED((n, d), _f32),
          pltpu.SemaphoreType.DMA,
      ],
  )
  def k(nh_hbm, sd_hbm, z128_hbm, outN_hbm, idxb, rows, accN, sem):
    c = lax.axis_index("c")
    s = lax.axis_index("s")
    w = s * _NC + c
    r0 = pl.multiple_of(s * rps, 8)
    pl.when(s < _NS - 1)(lambda: pltpu.sync_copy(
        z128_hbm.at[pl.ds(r0, rps)], accN.at[pl.ds(r0, rps)]))
    pl.when(s == _NS - 1)(lambda: pltpu.sync_copy(
        z128_hbm.at[pl.ds(r0, rlast)], accN.at[pl.ds(r0, rlast)]))
    plsc.subcore_barrier()
    lo = nchunks * w // _NW
    hi = nchunks * (w + 1) // _NW

    def step(i, carry):
      pltpu.sync_copy(sd_hbm.at[i], idxb)
      pltpu.async_copy(nh_hbm.at[idxb.at[0]], rows, sem).wait()
      pltpu.sync_copy(rows, accN.at[idxb.at[1]], add=True)
      return carry

    lax.fori_loop(lo, hi, step, 0)
    plsc.subcore_barrier()
    pl.when(s < _NS - 1)(lambda: pltpu.sync_copy(
        accN.at[pl.ds(r0, rps)], outN_hbm.at[c, pl.ds(r0, rps)]))
    pl.when(s == _NS - 1)(lambda: pltpu.sync_copy(
        accN.at[pl.ds(r0, rlast)], outN_hbm.at[c, pl.ds(r0, rlast)]))

  return k(nh, sd, z128)


def _layer(nh, sp, W0, W1, seaug=None, wfaug=None):
  n, d = nh.shape
  bn = 1000
  grid = n // bn
  with_edge = seaug is not None

  def body(*refs):
    if with_edge:
      nh_ref, sp_ref, w0_ref, w1_ref, se_ref, wf_ref, out_ref = refs
    else:
      nh_ref, sp_ref, w0_ref, w1_ref, out_ref = refs
    x = nh_ref[...]
    sagg = sp_ref[0] + sp_ref[1]
    acc = jnp.dot(x, w0_ref[...], preferred_element_type=_f32, precision=lax.Precision.HIGHEST)
    acc = acc + jnp.dot(sagg, w1_ref[...], preferred_element_type=_f32, precision=lax.Precision.HIGHEST)
    if with_edge:
      se = se_ref[0] + se_ref[1]
      acc = acc + jnp.dot(se, wf_ref[...], preferred_element_type=_f32, precision=lax.Precision.HIGHEST)
    out_ref[...] = jnp.maximum(acc, 0.0)

  in_specs = [
      pl.BlockSpec((bn, d), lambda i: (i, 0)),
      pl.BlockSpec((2, bn, d), lambda i: (0, i, 0)),
      _whole((d, d)), _whole((d, d)),
  ]
  args = [nh, sp, W0, W1]
  if with_edge:
    in_specs += [pl.BlockSpec((2, bn, d), lambda i: (0, i, 0)),
                 _whole((d, d))]
    args += [seaug, wfaug]

  return pl.pallas_call(
      body,
      grid=(grid,),
      in_specs=in_specs,
      out_specs=pl.BlockSpec((bn, d), lambda i: (i, 0)),
      out_shape=jax.ShapeDtypeStruct((n, d), _f32),
  )(*args)


def _pool(nh, node_index):
  n, d = nh.shape
  g = _G

  def body(idx_ref, nh_ref, out_ref, sum_scr, mx_scr, cnt_scr):
    sum_scr[...] = jnp.zeros((g, d), _f32)
    mx_scr[...] = jnp.full((g, d), -jnp.inf, _f32)
    cnt_scr[...] = jnp.zeros((g, d), _f32)

    def step(i, carry):
      gi = idx_ref[i]
      row = nh_ref[pl.ds(i, 1), :]
      sum_scr[pl.ds(gi, 1), :] = sum_scr[pl.ds(gi, 1), :] + row
      mx_scr[pl.ds(gi, 1), :] = jnp.maximum(mx_scr[pl.ds(gi, 1), :], row)
      cnt_scr[pl.ds(gi, 1), :] = cnt_scr[pl.ds(gi, 1), :] + 1.0
      return carry

    lax.fori_loop(0, n, step, 0)
    cnt = cnt_scr[:, 0:1]
    mean = sum_scr[...] / jnp.maximum(cnt, 1.0)
    mx = jnp.where(cnt > 0.0, mx_scr[...], 0.0)
    out_ref[:, :d] = mean
    out_ref[:, d:] = mx

  return pl.pallas_call(
      body,
      in_specs=[
          pl.BlockSpec(memory_space=pltpu.SMEM),
          pl.BlockSpec((n, d), lambda: (0, 0)),
      ],
      out_specs=pl.BlockSpec((g, 2 * d), lambda: (0, 0)),
      out_shape=jax.ShapeDtypeStruct((g, 2 * d), _f32),
      scratch_shapes=[
          pltpu.VMEM((g, d), _f32),
          pltpu.VMEM((g, d), _f32),
          pltpu.VMEM((g, d), _f32),
      ],
  )(node_index, nh)


def _sc_gather(pool, nh, edge8, quad):
  g, d2 = pool.shape
  n, d = nh.shape
  d8 = edge8.shape[1]
  nchunks = quad.shape[0]
  cp = nchunks * _CHUNK

  @functools.partial(
      pl.kernel, mesh=_sc_mesh(),
      out_type=(jax.ShapeDtypeStruct((cp, d2), _f32),
                jax.ShapeDtypeStruct((cp, d), _f32),
                jax.ShapeDtypeStruct((cp, d), _f32),
                jax.ShapeDtypeStruct((cp, d8), _f32)),
      scratch_types=[
          pltpu.VMEM((4, _CHUNK), jnp.int32),
          pltpu.VMEM((_CHUNK, d2), _f32),
          pltpu.VMEM((_CHUNK, d), _f32),
          pltpu.VMEM((_CHUNK, d), _f32),
          pltpu.VMEM((_CHUNK, d8), _f32),
          pltpu.SemaphoreType.DMA,
      ],
  )
  def k(pool_hbm, nh_hbm, edge_hbm, quad_hbm,
        outP_hbm, out0_hbm, out1_hbm, outE_hbm,
        idxq, pbuf, b0, b1, ebuf, sem):
    c = lax.axis_index("c")
    s = lax.axis_index("s")
    w = s * _NC + c
    lo = nchunks * w // _NW
    hi = nchunks * (w + 1) // _NW

    def step(i, carry):
      pltpu.sync_copy(quad_hbm.at[i], idxq)
      d0 = pltpu.async_copy(nh_hbm.at[idxq.at[0]], b0, sem)
      d1 = pltpu.async_copy(nh_hbm.at[idxq.at[1]], b1, sem)
      d2 = pltpu.async_copy(pool_hbm.at[idxq.at[2]], pbuf, sem)
      d3 = pltpu.async_copy(edge_hbm.at[idxq.at[3]], ebuf, sem)
      d0.wait()
      d1.wait()
      d2.wait()
      d3.wait()
      base = i * _CHUNK
      pltpu.sync_copy(pbuf, outP_hbm.at[pl.ds(base, _CHUNK)])
      pltpu.sync_copy(b0, out0_hbm.at[pl.ds(base, _CHUNK)])
      pltpu.sync_copy(b1, out1_hbm.at[pl.ds(base, _CHUNK)])
      pltpu.sync_copy(ebuf, outE_hbm.at[pl.ds(base, _CHUNK)])
      return carry

    lax.fori_loop(lo, hi, step, 0)

  return k(pool, nh, edge8, quad)


def _head(Pc, N0, N1, EC8, celo, Wp, Wn0, Wn1, Wat, WeH, b1eff,
          p1g, p1bt, p2W, p2b, p2g, p2bt, p3T, p3b, de):
  cp, d2 = Pc.shape
  d = N0.shape[1]
  h1 = Wp.shape[1]
  h2 = p2W.shape[1]
  bt = 256
  grid = cp // bt
  ngroup = EC8.shape[1] // de

  def body(pc_ref, n0_ref, n1_ref, ec_ref, celo_ref, wp_ref, wn0_ref, wn1_ref,
           wat_ref, weh_ref, b1_ref, g1_ref, bt1_ref, w2_ref, b2_ref, g2_ref,
           bt2_ref, p3_ref, p3b_ref, out_ref):
    n0 = n0_ref[...]
    n1 = n1_ref[...]
    att = n0 + n1 - n0 * n1
    ec8 = ec_ref[...]
    cem = celo_ref[...]
    ec = jnp.zeros((bt, de), _f32)
    for kk in range(ngroup):
      m = (cem == kk).astype(_f32)[:, None]
      ec = ec + m * ec8[:, kk * de:(kk + 1) * de]
    h = jnp.dot(pc_ref[...], wp_ref[...], preferred_element_type=_f32, precision=lax.Precision.HIGHEST)
    h = h + jnp.dot(n0, wn0_ref[...], preferred_element_type=_f32, precision=lax.Precision.HIGHEST)
    h = h + jnp.dot(n1, wn1_ref[...], preferred_element_type=_f32, precision=lax.Precision.HIGHEST)
    h = h + jnp.dot(att, wat_ref[...], preferred_element_type=_f32, precision=lax.Precision.HIGHEST)
    h = h + jnp.dot(ec, weh_ref[...], preferred_element_type=_f32, precision=lax.Precision.HIGHEST)
    h = h + b1_ref[0][None, :]
    h = h * (_RS * g1_ref[...])[None, :] + bt1_ref[...][None, :]
    h = jnp.maximum(h, 0.0)
    h = jnp.dot(h, w2_ref[...], preferred_element_type=_f32, precision=lax.Precision.HIGHEST) + b2_ref[...][None, :]
    h = h * (_RS * g2_ref[...])[None, :] + bt2_ref[...][None, :]
    h = jnp.maximum(h, 0.0)
    res = jnp.sum(h * p3_ref[0][None, :], axis=1) + p3b_ref[0]
    out_ref[0, 0, :] = res

  return pl.pallas_call(
      body,
      grid=(grid,),
      in_specs=[
          pl.BlockSpec((bt, d2), lambda i: (i, 0)),
          pl.BlockSpec((bt, d), lambda i: (i, 0)),
          pl.BlockSpec((bt, d), lambda i: (i, 0)),
          pl.BlockSpec((bt, EC8.shape[1]), lambda i: (i, 0)),
          pl.BlockSpec((bt,), lambda i: (i,)),
          _whole((d2, h1)), _whole((d, h1)), _whole((d, h1)), _whole((d, h1)),
          _whole((de, h1)), _whole((8, h1)), _whole((h1,)), _whole((h1,)),
          _whole((h1, h2)), _whole((h2,)), _whole((h2,)), _whole((h2,)),
          _whole((8, h2)),
          pl.BlockSpec(memory_space=pltpu.SMEM),
      ],
      out_specs=pl.BlockSpec((1, 1, bt), lambda i: (i, 0, 0)),
      out_shape=jax.ShapeDtypeStruct((grid, 1, bt), _f32),
  )(Pc, N0, N1, EC8, celo, Wp, Wn0, Wn1, Wat, WeH, b1eff,
    p1g, p1bt, p2W, p2b, p2g, p2bt, p3T, p3b)


def kernel(node, edge, edge_index, node_index, coupling_index,
           W_nemb, b_nemb, W_eemb, b_eemb,
           enc1_W0, enc1_W1, enc1_W2,
           enc2_W0, enc2_W1, enc3_W0, enc3_W1,
           p1_W, p1_b, p1_g, p1_beta,
           p2_W, p2_b, p2_g, p2_beta,
           p3_W, p3_b):
  n, d = node.shape
  e, de = edge.shape
  c = coupling_index.shape[0]

  src = edge_index[:, 0]
  dst = edge_index[:, 1]
  ne_chunks = e // _CHUNK
  sd = jnp.stack([src.reshape(ne_chunks, _CHUNK),
                  dst.reshape(ne_chunks, _CHUNK)], axis=1)
  z128 = jnp.zeros((n, d), _f32)
  edge_aug = jnp.concatenate(
      [edge, jnp.ones((e, 1), _f32), jnp.zeros((e, d - de - 1), _f32)], axis=1)
  edge8 = edge.reshape(e // 8, 8 * de)

  nh0, wfaug, weh, b1eff = _prep(
      node, W_nemb, b_nemb, W_eemb, b_eemb, enc1_W2, p1_W[5 * d:6 * d], p1_b)

  seaug = _sc_edge_prep(sd, edge_aug, z128, n)
  s1 = _sc_scatter(nh0, sd, z128)
  nh1 = _layer(nh0, s1, enc1_W0, enc1_W1, seaug, wfaug)
  s2 = _sc_scatter(nh1, sd, z128)
  nh2 = _layer(nh1, s2, enc2_W0, enc2_W1)
  s3 = _sc_scatter(nh2, sd, z128)
  nh3 = _layer(nh2, s3, enc3_W0, enc3_W1)

  pool = _pool(nh3, node_index)

  nc_chunks = -(-c // _CHUNK)
  cpad = nc_chunks * _CHUNK - c
  a0 = coupling_index[:, 0]
  a1 = coupling_index[:, 1]
  cb = coupling_index[:, 3]
  ce = coupling_index[:, 4]
  quad = jnp.stack([
      jnp.pad(a0, (0, cpad)).reshape(nc_chunks, _CHUNK),
      jnp.pad(a1, (0, cpad)).reshape(nc_chunks, _CHUNK),
      jnp.pad(cb, (0, cpad)).reshape(nc_chunks, _CHUNK),
      jnp.pad(ce // 8, (0, cpad)).reshape(nc_chunks, _CHUNK),
  ], axis=1)
  celo = jnp.pad(ce % 8, (0, cpad))
  Pc, N0, N1, EC8 = _sc_gather(pool, nh3, edge8, quad)

  out = _head(Pc, N0, N1, EC8, celo,
              p1_W[0:2 * d], p1_W[2 * d:3 * d], p1_W[3 * d:4 * d],
              p1_W[4 * d:5 * d], weh, b1eff,
              p1_g, p1_beta, p2_W, p2_b, p2_g, p2_beta,
              jnp.broadcast_to(p3_W.T, (8, p3_W.shape[0])), p3_b, de)
  return out.reshape(-1)[:c]


# speedup vs baseline: 3.3680x; 2.0156x over previous
import functools

import jax
import jax.numpy as jnp
from jax import lax
from jax.experimental import pallas as pl
from jax.experimental.pallas import tpu as pltpu
from jax.experimental.pallas import tpu_sc as plsc

_NC, _NS = 2, 16
_NW = _NC * _NS
_CHUNK = 128
_G = 512
_RS = float((1.0 + 1e-5) ** -0.5)
_f32 = jnp.float32


def _whole(shape):
  return pl.BlockSpec(shape, lambda i: (0,) * len(shape))


def _prep(node, W_nemb, b_nemb, W_eemb, b_eemb, e1W2, p1We, p1b):
  n, d = node.shape
  de = W_eemb.shape[0]
  h1 = p1We.shape[1]
  bn = 1000
  grid = n // bn

  def body(node_ref, wn_ref, bnemb_ref, we_ref, beemb_ref, e1w2_ref, p1we_ref,
           p1b_ref, nh0_ref, wfaug_ref, weh_ref, b1eff_ref):
    x = node_ref[...]
    nh0_ref[...] = (jnp.dot(x, wn_ref[...], preferred_element_type=_f32, precision=lax.Precision.HIGHEST)
                    + bnemb_ref[...][None, :])

    @pl.when(pl.program_id(0) == 0)
    def _():
      we = we_ref[...]
      e1w2 = e1w2_ref[...]
      p1we = p1we_ref[...]
      be = beemb_ref[...]
      wfaug_ref[...] = jnp.zeros((d, d), _f32)
      wfaug_ref[0:de, :] = jnp.dot(we, e1w2, preferred_element_type=_f32, precision=lax.Precision.HIGHEST)
      bf = jnp.sum(be[:, None] * e1w2, axis=0)
      wfaug_ref[de:de + 1, :] = bf[None, :]
      weh_ref[...] = jnp.dot(we, p1we, preferred_element_type=_f32, precision=lax.Precision.HIGHEST)
      b1 = p1b_ref[...] + jnp.sum(be[:, None] * p1we, axis=0)
      b1eff_ref[...] = jnp.broadcast_to(b1[None, :], (8, h1))

  return pl.pallas_call(
      body,
      grid=(grid,),
      in_specs=[
          pl.BlockSpec((bn, d), lambda i: (i, 0)),
          _whole((d, d)), _whole((d,)), _whole((de, d)), _whole((d,)),
          _whole((d, d)), _whole((d, h1)), _whole((h1,)),
      ],
      out_specs=(
          pl.BlockSpec((bn, d), lambda i: (i, 0)),
          _whole((d, d)), _whole((de, h1)), _whole((8, h1)),
      ),
      out_shape=(
          jax.ShapeDtypeStruct((n, d), _f32),
          jax.ShapeDtypeStruct((d, d), _f32),
          jax.ShapeDtypeStruct((de, h1), _f32),
          jax.ShapeDtypeStruct((8, h1), _f32),
      ),
  )(node, W_nemb, b_nemb, W_eemb, b_eemb, e1W2, p1We, p1b)


def _sc_mesh():
  return plsc.VectorSubcoreMesh(core_axis_name="c", subcore_axis_name="s",
                                num_cores=_NC, num_subcores=_NS)


def _rows_split(n):
  rps = (-(-n // _NS) + 7) // 8 * 8
  last = n - (_NS - 1) * rps
  return rps, last


def _sc_edge_prep(sd, edge_aug, z128, n):
  d = edge_aug.shape[1]
  nchunks = sd.shape[0]
  rps, rlast = _rows_split(n)

  @functools.partial(
      pl.kernel, mesh=_sc_mesh(),
      out_type=jax.ShapeDtypeStruct((_NC, n, d), _f32),
      scratch_types=[
          pltpu.VMEM((2, _CHUNK), jnp.int32),
          pltpu.VMEM((_CHUNK, d), _f32),
          pltpu.VMEM_SHARED((n, d), _f32),
      ],
  )
  def k(sd_hbm, ea_hbm, z128_hbm, out_hbm, idxb, erows, acc):
    c = lax.axis_index("c")
    s = lax.axis_index("s")
    w = s * _NC + c
    r0 = pl.multiple_of(s * rps, 8)
    pl.when(s < _NS - 1)(lambda: pltpu.sync_copy(
        z128_hbm.at[pl.ds(r0, rps)], acc.at[pl.ds(r0, rps)]))
    pl.when(s == _NS - 1)(lambda: pltpu.sync_copy(
        z128_hbm.at[pl.ds(r0, rlast)], acc.at[pl.ds(r0, rlast)]))
    plsc.subcore_barrier()
    lo = nchunks * w // _NW
    hi = nchunks * (w + 1) // _NW

    def step(i, carry):
      pltpu.sync_copy(sd_hbm.at[i], idxb)
      pltpu.sync_copy(ea_hbm.at[pl.ds(i * _CHUNK, _CHUNK)], erows)
      pltpu.sync_copy(erows, acc.at[idxb.at[1]], add=True)
      return carry

    lax.fori_loop(lo, hi, step, 0)
    plsc.subcore_barrier()
    pl.when(s < _NS - 1)(lambda: pltpu.sync_copy(
        acc.at[pl.ds(r0, rps)], out_hbm.at[c, pl.ds(r0, rps)]))
    pl.when(s == _NS - 1)(lambda: pltpu.sync_copy(
        acc.at[pl.ds(r0, rlast)], out_hbm.at[c, pl.ds(r0, rlast)]))

  return k(sd, edge_aug, z128)


def _sc_scatter(nh, sd, z128):
  n, d = nh.shape
  nchunks = sd.shape[0]
  rps, rlast = _rows_split(n)

  @functools.partial(
      pl.kernel, mesh=_sc_mesh(),
      out_type=jax.ShapeDtypeStruct((_NC, n, d), _f32),
      scratch_types=[
          pltpu.VMEM((2, _CHUNK), jnp.int32),
          pltpu.VMEM((_CHUNK, d), _f32),
          pltpu.VMEM_SHARED((n, d), _f32),
          pltpu.SemaphoreType.DMA,
      ],
  )
  def k(nh_hbm, sd_hbm, z128_hbm, outN_hbm, idxb, rows, accN, sem):
    c = lax.axis_index("c")
    s = lax.axis_index("s")
    w = s * _NC + c
    r0 = pl.multiple_of(s * rps, 8)
    pl.when(s < _NS - 1)(lambda: pltpu.sync_copy(
        z128_hbm.at[pl.ds(r0, rps)], accN.at[pl.ds(r0, rps)]))
    pl.when(s == _NS - 1)(lambda: pltpu.sync_copy(
        z128_hbm.at[pl.ds(r0, rlast)], accN.at[pl.ds(r0, rlast)]))
    plsc.subcore_barrier()
    lo = nchunks * w // _NW
    hi = nchunks * (w + 1) // _NW

    def step(i, carry):
      pltpu.sync_copy(sd_hbm.at[i], idxb)
      pltpu.async_copy(nh_hbm.at[idxb.at[0]], rows, sem).wait()
      pltpu.sync_copy(rows, accN.at[idxb.at[1]], add=True)
      return carry

    lax.fori_loop(lo, hi, step, 0)
    plsc.subcore_barrier()
    pl.when(s < _NS - 1)(lambda: pltpu.sync_copy(
        accN.at[pl.ds(r0, rps)], outN_hbm.at[c, pl.ds(r0, rps)]))
    pl.when(s == _NS - 1)(lambda: pltpu.sync_copy(
        accN.at[pl.ds(r0, rlast)], outN_hbm.at[c, pl.ds(r0, rlast)]))

  return k(nh, sd, z128)


def _layer(nh, sp, W0, W1, seaug=None, wfaug=None):
  n, d = nh.shape
  bn = 1000
  grid = n // bn
  with_edge = seaug is not None

  def body(*refs):
    if with_edge:
      nh_ref, sp_ref, w0_ref, w1_ref, se_ref, wf_ref, out_ref = refs
    else:
      nh_ref, sp_ref, w0_ref, w1_ref, out_ref = refs
    x = nh_ref[...]
    sagg = sp_ref[0] + sp_ref[1]
    acc = jnp.dot(x, w0_ref[...], preferred_element_type=_f32, precision=lax.Precision.HIGHEST)
    acc = acc + jnp.dot(sagg, w1_ref[...], preferred_element_type=_f32, precision=lax.Precision.HIGHEST)
    if with_edge:
      se = se_ref[0] + se_ref[1]
      acc = acc + jnp.dot(se, wf_ref[...], preferred_element_type=_f32, precision=lax.Precision.HIGHEST)
    out_ref[...] = jnp.maximum(acc, 0.0)

  in_specs = [
      pl.BlockSpec((bn, d), lambda i: (i, 0)),
      pl.BlockSpec((2, bn, d), lambda i: (0, i, 0)),
      _whole((d, d)), _whole((d, d)),
  ]
  args = [nh, sp, W0, W1]
  if with_edge:
    in_specs += [pl.BlockSpec((2, bn, d), lambda i: (0, i, 0)),
                 _whole((d, d))]
    args += [seaug, wfaug]

  return pl.pallas_call(
      body,
      grid=(grid,),
      in_specs=in_specs,
      out_specs=pl.BlockSpec((bn, d), lambda i: (i, 0)),
      out_shape=jax.ShapeDtypeStruct((n, d), _f32),
  )(*args)


def _pool(nh, node_index):
  n, d = nh.shape
  g = _G

  def body(idx_ref, nh_ref, out_ref, sum_scr, mx_scr, cnt_scr):
    sum_scr[...] = jnp.zeros((g, d), _f32)
    mx_scr[...] = jnp.full((g, d), -jnp.inf, _f32)
    cnt_scr[...] = jnp.zeros((g, d), _f32)

    def step(i, carry):
      gi = idx_ref[i]
      row = nh_ref[pl.ds(i, 1), :]
      sum_scr[pl.ds(gi, 1), :] = sum_scr[pl.ds(gi, 1), :] + row
      mx_scr[pl.ds(gi, 1), :] = jnp.maximum(mx_scr[pl.ds(gi, 1), :], row)
      cnt_scr[pl.ds(gi, 1), :] = cnt_scr[pl.ds(gi, 1), :] + 1.0
      return carry

    lax.fori_loop(0, n, step, 0)
    cnt = cnt_scr[:, 0:1]
    mean = sum_scr[...] / jnp.maximum(cnt, 1.0)
    mx = jnp.where(cnt > 0.0, mx_scr[...], 0.0)
    out_ref[:, :d] = mean
    out_ref[:, d:] = mx

  return pl.pallas_call(
      body,
      in_specs=[
          pl.BlockSpec(memory_space=pltpu.SMEM),
          pl.BlockSpec((n, d), lambda: (0, 0)),
      ],
      out_specs=pl.BlockSpec((g, 2 * d), lambda: (0, 0)),
      out_shape=jax.ShapeDtypeStruct((g, 2 * d), _f32),
      scratch_shapes=[
          pltpu.VMEM((g, d), _f32),
          pltpu.VMEM((g, d), _f32),
          pltpu.VMEM((g, d), _f32),
      ],
  )(node_index, nh)


def _sc_gather(pool, nh, edge8, quad):
  g, d2 = pool.shape
  n, d = nh.shape
  d8 = edge8.shape[1]
  nchunks = quad.shape[0]
  cp = nchunks * _CHUNK

  @functools.partial(
      pl.kernel, mesh=_sc_mesh(),
      out_type=(jax.ShapeDtypeStruct((cp, d2), _f32),
                jax.ShapeDtypeStruct((cp, d), _f32),
                jax.ShapeDtypeStruct((cp, d), _f32),
                jax.ShapeDtypeStruct((cp, d8), _f32)),
      scratch_types=[
          pltpu.VMEM((4, _CHUNK), jnp.int32),
          pltpu.VMEM((_CHUNK, d2), _f32),
          pltpu.VMEM((_CHUNK, d), _f32),
          pltpu.VMEM((_CHUNK, d), _f32),
          pltpu.VMEM((_CHUNK, d8), _f32),
          pltpu.SemaphoreType.DMA,
      ],
  )
  def k(pool_hbm, nh_hbm, edge_hbm, quad_hbm,
        outP_hbm, out0_hbm, out1_hbm, outE_hbm,
        idxq, pbuf, b0, b1, ebuf, sem):
    c = lax.axis_index("c")
    s = lax.axis_index("s")
    w = s * _NC + c
    lo = nchunks * w // _NW
    hi = nchunks * (w + 1) // _NW

    def step(i, carry):
      pltpu.sync_copy(quad_hbm.at[i], idxq)
      d0 = pltpu.async_copy(nh_hbm.at[idxq.at[0]], b0, sem)
      d1 = pltpu.async_copy(nh_hbm.at[idxq.at[1]], b1, sem)
      d2 = pltpu.async_copy(pool_hbm.at[idxq.at[2]], pbuf, sem)
      d3 = pltpu.async_copy(edge_hbm.at[idxq.at[3]], ebuf, sem)
      d0.wait()
      d1.wait()
      d2.wait()
      d3.wait()
      base = i * _CHUNK
      pltpu.sync_copy(pbuf, outP_hbm.at[pl.ds(base, _CHUNK)])
      pltpu.sync_copy(b0, out0_hbm.at[pl.ds(base, _CHUNK)])
      pltpu.sync_copy(b1, out1_hbm.at[pl.ds(base, _CHUNK)])
      pltpu.sync_copy(ebuf, outE_hbm.at[pl.ds(base, _CHUNK)])
      return carry

    lax.fori_loop(lo, hi, step, 0)

  return k(pool, nh, edge8, quad)


def _head(Pc, N0, N1, EC8, celo, Wp, Wn0, Wn1, Wat, WeH, b1eff,
          p1g, p1bt, p2W, p2b, p2g, p2bt, p3T, p3b, de):
  cp, d2 = Pc.shape
  d = N0.shape[1]
  h1 = Wp.shape[1]
  h2 = p2W.shape[1]
  bt = 256
  grid = cp // bt
  ngroup = EC8.shape[1] // de

  def body(pc_ref, n0_ref, n1_ref, ec_ref, celo_ref, wp_ref, wn0_ref, wn1_ref,
           wat_ref, weh_ref, b1_ref, g1_ref, bt1_ref, w2_ref, b2_ref, g2_ref,
           bt2_ref, p3_ref, p3b_ref, out_ref):
    n0 = n0_ref[...]
    n1 = n1_ref[...]
    att = n0 + n1 - n0 * n1
    ec8 = ec_ref[...]
    cem = celo_ref[...]
    ec = jnp.zeros((bt, de), _f32)
    for kk in range(ngroup):
      m = (cem == kk).astype(_f32)[:, None]
      ec = ec + m * ec8[:, kk * de:(kk + 1) * de]
    h = jnp.dot(pc_ref[...], wp_ref[...], preferred_element_type=_f32, precision=lax.Precision.DEFAULT)
    h = h + jnp.dot(n0, wn0_ref[...], preferred_element_type=_f32, precision=lax.Precision.DEFAULT)
    h = h + jnp.dot(n1, wn1_ref[...], preferred_element_type=_f32, precision=lax.Precision.DEFAULT)
    h = h + jnp.dot(att, wat_ref[...], preferred_element_type=_f32, precision=lax.Precision.DEFAULT)
    h = h + jnp.dot(ec, weh_ref[...], preferred_element_type=_f32, precision=lax.Precision.DEFAULT)
    h = h + b1_ref[0][None, :]
    h = h * (_RS * g1_ref[...])[None, :] + bt1_ref[...][None, :]
    h = jnp.maximum(h, 0.0)
    h = jnp.dot(h, w2_ref[...], preferred_element_type=_f32, precision=lax.Precision.DEFAULT) + b2_ref[...][None, :]
    h = h * (_RS * g2_ref[...])[None, :] + bt2_ref[...][None, :]
    h = jnp.maximum(h, 0.0)
    res = jnp.sum(h * p3_ref[0][None, :], axis=1) + p3b_ref[0]
    out_ref[0, 0, :] = res

  return pl.pallas_call(
      body,
      grid=(grid,),
      in_specs=[
          pl.BlockSpec((bt, d2), lambda i: (i, 0)),
          pl.BlockSpec((bt, d), lambda i: (i, 0)),
          pl.BlockSpec((bt, d), lambda i: (i, 0)),
          pl.BlockSpec((bt, EC8.shape[1]), lambda i: (i, 0)),
          pl.BlockSpec((bt,), lambda i: (i,)),
          _whole((d2, h1)), _whole((d, h1)), _whole((d, h1)), _whole((d, h1)),
          _whole((de, h1)), _whole((8, h1)), _whole((h1,)), _whole((h1,)),
          _whole((h1, h2)), _whole((h2,)), _whole((h2,)), _whole((h2,)),
          _whole((8, h2)),
          pl.BlockSpec(memory_space=pltpu.SMEM),
      ],
      out_specs=pl.BlockSpec((1, 1, bt), lambda i: (i, 0, 0)),
      out_shape=jax.ShapeDtypeStruct((grid, 1, bt), _f32),
  )(Pc, N0, N1, EC8, celo, Wp, Wn0, Wn1, Wat, WeH, b1eff,
    p1g, p1bt, p2W, p2b, p2g, p2bt, p3T, p3b)


def kernel(node, edge, edge_index, node_index, coupling_index,
           W_nemb, b_nemb, W_eemb, b_eemb,
           enc1_W0, enc1_W1, enc1_W2,
           enc2_W0, enc2_W1, enc3_W0, enc3_W1,
           p1_W, p1_b, p1_g, p1_beta,
           p2_W, p2_b, p2_g, p2_beta,
           p3_W, p3_b):
  n, d = node.shape
  e, de = edge.shape
  c = coupling_index.shape[0]

  src = edge_index[:, 0]
  dst = edge_index[:, 1]
  ne_chunks = e // _CHUNK
  sd = jnp.stack([src.reshape(ne_chunks, _CHUNK),
                  dst.reshape(ne_chunks, _CHUNK)], axis=1)
  z128 = jnp.zeros((n, d), _f32)
  edge_aug = jnp.concatenate(
      [edge, jnp.ones((e, 1), _f32), jnp.zeros((e, d - de - 1), _f32)], axis=1)
  edge8 = edge.reshape(e // 8, 8 * de)

  nh0, wfaug, weh, b1eff = _prep(
      node, W_nemb, b_nemb, W_eemb, b_eemb, enc1_W2, p1_W[5 * d:6 * d], p1_b)

  seaug = _sc_edge_prep(sd, edge_aug, z128, n)
  s1 = _sc_scatter(nh0, sd, z128)
  nh1 = _layer(nh0, s1, enc1_W0, enc1_W1, seaug, wfaug)
  s2 = _sc_scatter(nh1, sd, z128)
  nh2 = _layer(nh1, s2, enc2_W0, enc2_W1)
  s3 = _sc_scatter(nh2, sd, z128)
  nh3 = _layer(nh2, s3, enc3_W0, enc3_W1)

  pool = _pool(nh3, node_index)

  nc_chunks = -(-c // _CHUNK)
  cpad = nc_chunks * _CHUNK - c
  a0 = coupling_index[:, 0]
  a1 = coupling_index[:, 1]
  cb = coupling_index[:, 3]
  ce = coupling_index[:, 4]
  quad = jnp.stack([
      jnp.pad(a0, (0, cpad)).reshape(nc_chunks, _CHUNK),
      jnp.pad(a1, (0, cpad)).reshape(nc_chunks, _CHUNK),
      jnp.pad(cb, (0, cpad)).reshape(nc_chunks, _CHUNK),
      jnp.pad(ce // 8, (0, cpad)).reshape(nc_chunks, _CHUNK),
  ], axis=1)
  celo = jnp.pad(ce % 8, (0, cpad))
  Pc, N0, N1, EC8 = _sc_gather(pool, nh3, edge8, quad)

  out = _head(Pc, N0, N1, EC8, celo,
              p1_W[0:2 * d], p1_W[2 * d:3 * d], p1_W[3 * d:4 * d],
              p1_W[4 * d:5 * d], weh, b1eff,
              p1_g, p1_beta, p2_W, p2_b, p2_g, p2_beta,
              jnp.broadcast_to(p3_W.T, (8, p3_W.shape[0])), p3_b, de)
  return out.reshape(-1)[:c]
